# Initial kernel scaffold; baseline (speedup 1.0000x reference)
import jax
import jax.numpy as jnp
from jax.experimental import pallas as pl


def kernel(ze, embedding):
    raise NotImplementedError("write your pallas kernel here")


# speedup vs baseline: 1.0355x; 1.0355x over previous
import jax
import jax.numpy as jnp
from jax.experimental import pallas as pl
from jax.experimental.pallas import tpu as pltpu

_NE = 1024
_D = 64
_P = 1024
_B = 16
_BETA = 0.25


def _vq_body(ze_ref, emb_ref, embt_ref, st_ref, idx_ref, loss_ref):
    b = pl.program_id(0)
    ze = ze_ref[0]
    emb = emb_ref[...]
    zs = jnp.sum(ze * ze, axis=0, keepdims=True)
    es = jnp.sum(emb * emb, axis=1, keepdims=True)
    m = jnp.dot(emb, ze, preferred_element_type=jnp.float32)
    d = (zs + es) - 2.0 * m
    minv = jnp.min(d, axis=0, keepdims=True)
    iota = jax.lax.broadcasted_iota(jnp.int32, (_NE, _P), 0)
    cand = jnp.where(d == minv, iota, jnp.int32(_NE))
    idx = jnp.min(cand, axis=0, keepdims=True)
    idx_ref[0] = idx
    onehot = (iota == idx).astype(jnp.float32)
    zq = jnp.dot(embt_ref[...], onehot, preferred_element_type=jnp.float32)
    diff = zq - ze
    st_ref[0] = ze + diff
    part = jnp.sum(diff * diff).reshape(1, 1)

    @pl.when(b == 0)
    def _():
        loss_ref[...] = part

    @pl.when(b != 0)
    def _():
        loss_ref[...] = loss_ref[...] + part


def kernel(ze, embedding):
    B, L, H, W = ze.shape
    ze_r = ze.reshape(B, L, H * W)
    embt = embedding.T

    st, idx, loss_sum = pl.pallas_call(
        _vq_body,
        grid=(B,),
        in_specs=[
            pl.BlockSpec((1, L, H * W), lambda b: (b, 0, 0)),
            pl.BlockSpec((_NE, _D), lambda b: (0, 0)),
            pl.BlockSpec((_D, _NE), lambda b: (0, 0)),
        ],
        out_specs=[
            pl.BlockSpec((1, L, H * W), lambda b: (b, 0, 0)),
            pl.BlockSpec((1, 1, _P), lambda b: (b, 0, 0)),
            pl.BlockSpec((1, 1), lambda b: (0, 0)),
        ],
        out_shape=[
            jax.ShapeDtypeStruct((B, L, H * W), jnp.float32),
            jax.ShapeDtypeStruct((B, 1, _P), jnp.int32),
            jax.ShapeDtypeStruct((1, 1), jnp.float32),
        ],
    )(ze_r, embedding, embt)

    z_q_st = st.reshape(B, L, H, W)
    n = float(B * L * H * W)
    mean_sq = loss_sum[0, 0] / n
    loss = mean_sq + _BETA * mean_sq
    min_idx = idx.reshape(-1, 1)
    return (z_q_st, loss, min_idx)
